# Initial kernel scaffold; baseline (speedup 1.0000x reference)
import jax
import jax.numpy as jnp
from jax.experimental import pallas as pl


def kernel(hidden_flat, probs, indices, W1, b1, W2, b2):
    raise NotImplementedError("write your pallas kernel here")


# speedup vs baseline: 1.9402x; 1.9402x over previous
import functools

import jax
import jax.numpy as jnp
from jax.experimental import pallas as pl
from jax.experimental.pallas import tpu as pltpu

E, T, H, F, K = 8, 4096, 1024, 4096, 2
TB = 512
FB = 512
NT = T // TB
NF = F // FB

_INV_SQRT2 = 0.7071067811865476


def _gelu_exact(x):
    return 0.5 * x * (1.0 + jax.lax.erf(x * _INV_SQRT2))


def _dense_body(probs_ref, idx_ref, x_ref, w1_ref, b1_ref, w2_ref, b2_ref,
                out_ref, acc_ref):
    e = pl.program_id(0)
    fj = pl.program_id(1)
    tb = pl.program_id(2)

    row0 = tb * TB

    @pl.when((e == 0) & (fj == 0))
    def _zero():
        acc_ref[pl.ds(row0, TB), :] = jnp.zeros((TB, H), jnp.float32)

    p = probs_ref[:, :]
    ids = idx_ref[:, :]
    coeff = jnp.sum(p * (ids == e).astype(jnp.float32), axis=0)

    x = x_ref[pl.ds(row0, TB), :]
    w1 = w1_ref[0].astype(jnp.bfloat16)
    h = jnp.dot(x, w1, preferred_element_type=jnp.float32) + b1_ref[0]
    h = _gelu_exact(h).astype(jnp.bfloat16)
    w2 = w2_ref[0].astype(jnp.bfloat16)
    y = jnp.dot(h, w2, preferred_element_type=jnp.float32)

    contrib = coeff[:, None] * y

    @pl.when(fj == 0)
    def _bias2():
        acc_ref[pl.ds(row0, TB), :] += coeff[:, None] * b2_ref[0]

    acc_ref[pl.ds(row0, TB), :] += contrib

    @pl.when((e == E - 1) & (fj == NF - 1))
    def _emit():
        out_ref[:, :] = acc_ref[pl.ds(row0, TB), :]


def _out_index(e, fj, tb):
    last = (e == E - 1) & (fj == NF - 1)
    return (jnp.where(last, tb, 0), 0)


@jax.jit
def _moe_dense(hidden_flat, probs, indices, W1, b1, W2, b2):
    x_bf = hidden_flat.astype(jnp.bfloat16)
    probs_t = probs.T
    idx_t = indices.astype(jnp.int32).T
    b1r = b1.reshape(E, 1, F)
    b2r = b2.reshape(E, 1, H)

    grid = (E, NF, NT)
    out = pl.pallas_call(
        _dense_body,
        grid=grid,
        in_specs=[
            pl.BlockSpec((K, TB), lambda e, fj, tb: (0, tb)),
            pl.BlockSpec((K, TB), lambda e, fj, tb: (0, tb)),
            pl.BlockSpec((T, H), lambda e, fj, tb: (0, 0)),
            pl.BlockSpec((1, H, FB), lambda e, fj, tb: (e, 0, fj)),
            pl.BlockSpec((1, 1, FB), lambda e, fj, tb: (e, 0, fj)),
            pl.BlockSpec((1, FB, H), lambda e, fj, tb: (e, fj, 0)),
            pl.BlockSpec((1, 1, H), lambda e, fj, tb: (e, 0, 0)),
        ],
        out_specs=pl.BlockSpec((TB, H), _out_index),
        out_shape=jax.ShapeDtypeStruct((T, H), jnp.float32),
        scratch_shapes=[pltpu.VMEM((T, H), jnp.float32)],
        compiler_params=pltpu.CompilerParams(
            dimension_semantics=("arbitrary", "arbitrary", "arbitrary"),
        ),
    )(probs_t, idx_t, x_bf, W1, b1r, W2, b2r)
    return out


def kernel(hidden_flat, probs, indices, W1, b1, W2, b2):
    return _moe_dense(hidden_flat, probs, indices, W1, b1, W2, b2)


# speedup vs baseline: 2.9912x; 1.5417x over previous
import functools

import jax
import jax.numpy as jnp
from jax.experimental import pallas as pl
from jax.experimental.pallas import tpu as pltpu

E, T, H, F, K = 8, 4096, 1024, 4096, 2
A = T * K
B = 512
P = A + E * B
NB = P // B

_INV_SQRT2 = 0.7071067811865476


def _gelu_exact(x):
    return 0.5 * x * (1.0 + jax.lax.erf(x * _INV_SQRT2))


def _mlp_body(be_ref, x_ref, w1_ref, b1_ref, w2_ref, b2_ref, y_ref):
    x = x_ref[...].astype(jnp.bfloat16)
    h = jnp.dot(x, w1_ref[0], preferred_element_type=jnp.float32) + b1_ref[0]
    h = _gelu_exact(h).astype(jnp.bfloat16)
    y_ref[...] = (jnp.dot(h, w2_ref[0], preferred_element_type=jnp.float32)
                  + b2_ref[0])


def _grouped_mlp(x_sorted, block_expert, W1b, b1r, W2b, b2r):
    grid_spec = pltpu.PrefetchScalarGridSpec(
        num_scalar_prefetch=1,
        grid=(NB,),
        in_specs=[
            pl.BlockSpec((B, H), lambda i, be: (i, 0)),
            pl.BlockSpec((1, H, F), lambda i, be: (be[i], 0, 0)),
            pl.BlockSpec((1, 1, F), lambda i, be: (be[i], 0, 0)),
            pl.BlockSpec((1, F, H), lambda i, be: (be[i], 0, 0)),
            pl.BlockSpec((1, 1, H), lambda i, be: (be[i], 0, 0)),
        ],
        out_specs=pl.BlockSpec((B, H), lambda i, be: (i, 0)),
    )
    return pl.pallas_call(
        _mlp_body,
        grid_spec=grid_spec,
        out_shape=jax.ShapeDtypeStruct((P, H), jnp.float32),
        compiler_params=pltpu.CompilerParams(
            dimension_semantics=("arbitrary",),
        ),
    )(block_expert, x_sorted, W1b, b1r, W2b, b2r)


def _route_scaffold(hidden_flat, eid):
    onehot = jax.nn.one_hot(eid, E, dtype=jnp.int32)
    counts = jnp.sum(onehot, axis=0)
    rank = jnp.cumsum(onehot, axis=0) - onehot
    myrank = jnp.sum(rank * onehot, axis=1)
    padded = (counts + B - 1) // B * B
    ends = jnp.cumsum(padded)
    bases = ends - padded
    pos = bases[eid] + myrank
    tok = jnp.arange(A, dtype=jnp.int32) // K
    x_sorted = jnp.zeros((P, H), jnp.float32).at[pos].set(hidden_flat[tok])
    blk = jnp.arange(NB, dtype=jnp.int32) * B
    be = jnp.minimum(jnp.sum(blk[:, None] >= ends[None, :], axis=1), E - 1)
    return pos, x_sorted, be.astype(jnp.int32)


def _combine_scaffold(y_all, pos, probs):
    p = probs.reshape(A)
    y = y_all[pos] * p[:, None]
    return y.reshape(T, K, H).sum(axis=1)


@jax.jit
def _moe(hidden_flat, probs, indices, W1, b1, W2, b2):
    eid = indices.astype(jnp.int32).reshape(A)
    W1b = W1.astype(jnp.bfloat16)
    W2b = W2.astype(jnp.bfloat16)
    b1r = b1.reshape(E, 1, F)
    b2r = b2.reshape(E, 1, H)
    pos, x_sorted, be = _route_scaffold(hidden_flat, eid)
    y_all = _grouped_mlp(x_sorted, be, W1b, b1r, W2b, b2r)
    return _combine_scaffold(y_all, pos, probs)


def kernel(hidden_flat, probs, indices, W1, b1, W2, b2):
    return _moe(hidden_flat, probs, indices, W1, b1, W2, b2)


# speedup vs baseline: 3.7097x; 1.2402x over previous
import functools

import jax
import jax.numpy as jnp
from jax import lax
from jax.experimental import pallas as pl
from jax.experimental.pallas import tpu as pltpu
from jax.experimental.pallas import tpu_sc as plsc

E, T, H, F, K = 8, 4096, 1024, 4096, 2
A = T * K
B = 512
P = A + E * B
NB = P // B
NBPAD = 32

NW = 32
CHUNK = A // NW
RCH = 64
RCHD = 32
TOKW = T // NW

_INV_SQRT2 = 0.7071067811865476

_mesh = plsc.VectorSubcoreMesh(core_axis_name="c", subcore_axis_name="s")


def _wid():
    return lax.axis_index("s") * 2 + lax.axis_index("c")


def _lanes():
    return lax.broadcasted_iota(jnp.int32, (16,), 0)


def _vi(x):
    return jnp.full((16,), x, jnp.int32)


def _vf(x):
    return jnp.full((16,), x, jnp.float32)


_GDN = lax.GatherDimensionNumbers(
    offset_dims=(), collapsed_slice_dims=(0,), start_index_map=(0,))


def _lgather(x, idx):
    return lax.gather(x, idx[:, None], _GDN, (1,),
                      mode=lax.GatherScatterMode.PROMISE_IN_BOUNDS)


def _lanesum(x):
    lanes = _lanes()
    for d in (1, 2, 4, 8):
        x = x + _lgather(x, lanes ^ _vi(d))
    return x


def _prefix_incl(x):
    lanes = _lanes()
    zero = jnp.zeros((16,), x.dtype)
    for d in (1, 2, 4, 8):
        shifted = _lgather(x, jnp.maximum(lanes - _vi(d), _vi(0)))
        x = x + jnp.where(lanes >= _vi(d), shifted, zero)
    return x


@functools.partial(
    pl.kernel, mesh=_mesh,
    out_type=jax.ShapeDtypeStruct((NW, 16), jnp.int32),
    scratch_types=[pltpu.VMEM((CHUNK,), jnp.int32),
                   pltpu.VMEM((16,), jnp.int32)])
def _sc_hist(eid_hbm, hist_hbm, eid_v, cnt_v):
    wid = _wid()
    pltpu.sync_copy(eid_hbm.at[pl.ds(wid * CHUNK, CHUNK)], eid_v)
    lanes = _lanes()

    def body(j, cnt):
        v = eid_v[pl.ds(j * 16, 16)]
        for e in range(E):
            pc = _lanesum(jnp.where(v == _vi(e), _vi(1), _vi(0)))
            cnt = cnt + jnp.where(lanes == _vi(e), pc, _vi(0))
        return cnt

    cnt = lax.fori_loop(0, CHUNK // 16, body, jnp.zeros((16,), jnp.int32))
    cnt_v[...] = cnt
    pltpu.sync_copy(cnt_v, hist_hbm.at[wid])


@functools.partial(
    pl.kernel, mesh=_mesh,
    out_type=(jax.ShapeDtypeStruct((A,), jnp.int32),
              jax.ShapeDtypeStruct((NBPAD,), jnp.int32),
              jax.ShapeDtypeStruct((P, H), jnp.float32)),
    scratch_types=[pltpu.VMEM((CHUNK,), jnp.int32),
                   pltpu.VMEM((NW * 16,), jnp.int32),
                   pltpu.VMEM((CHUNK,), jnp.int32),
                   pltpu.VMEM((CHUNK,), jnp.int32),
                   pltpu.VMEM((CHUNK // RCHD, RCHD), jnp.int32),
                   pltpu.VMEM((RCHD, H), jnp.float32),
                   pltpu.VMEM((RCHD, H), jnp.float32),
                   pltpu.VMEM((NBPAD,), jnp.int32),
                   pltpu.SemaphoreType.DMA,
                   pltpu.SemaphoreType.DMA])
def _sc_dispatch(eid_hbm, hist_hbm, hidden_hbm, pos_hbm, be_hbm, xs_hbm,
                 eid_v, hist_v, pos_v, tok_v, pidx_v, rows0, rows1, be_v,
                 sem0, sem1):
    wid = _wid()
    lanes = _lanes()
    pltpu.sync_copy(eid_hbm.at[pl.ds(wid * CHUNK, CHUNK)], eid_v)
    pltpu.sync_copy(hist_hbm, hist_v)

    def hbody(w, carry):
        tot, mine = carry
        row = hist_v[pl.ds(w * 16, 16)]
        sel = jnp.where(w < wid, 1, 0)
        return tot + row, mine + row * _vi(sel)

    zeros16 = jnp.zeros((16,), jnp.int32)
    tot, mine = lax.fori_loop(0, NW, hbody, (zeros16, zeros16))

    padded = ((tot + _vi(B - 1)) >> 9) << 9
    ends = _prefix_incl(padded)
    bases = ends - padded
    cur0 = bases + mine

    def pbody(j, cur):
        v = eid_v[pl.ds(j * 16, 16)]
        pos = _vi(0)
        for e in range(E):
            m = v == _vi(e)
            m01 = jnp.where(m, _vi(1), _vi(0))
            r = _prefix_incl(m01)
            base_e = _lanesum(jnp.where(lanes == _vi(e), cur, _vi(0)))
            pos = jnp.where(m, base_e - _vi(1) + r, pos)
            pc = _lgather(r, _vi(15))
            cur = cur + jnp.where(lanes == _vi(e), pc, _vi(0))
        pos_v[pl.ds(j * 16, 16)] = pos
        return cur

    lax.fori_loop(0, CHUNK // 16, pbody, cur0)
    pltpu.sync_copy(pos_v, pos_hbm.at[pl.ds(wid * CHUNK, CHUNK)])

    def tbody(j, _):
        base = wid * CHUNK + j * 16
        tok_v[pl.ds(j * 16, 16)] = (_vi(base) + lanes) >> _vi(1)
        return 0

    lax.fori_loop(0, CHUNK // 16, tbody, 0)

    for cb in range(CHUNK // RCHD):
        for g in range(RCHD // 16):
            pidx_v[cb, pl.ds(g * 16, 16)] = pos_v[pl.ds(cb * RCHD + g * 16,
                                                        16)]

    nch = CHUNK // RCHD
    bufs = (rows0, rows1)
    sems = (sem0, sem1)
    pend = [None, None]
    pend[0] = pltpu.async_copy(
        hidden_hbm.at[tok_v.at[pl.ds(0, RCHD)]], rows0, sem0)
    for cb in range(nch):
        sl = cb % 2
        pend[sl].wait()
        sc = pltpu.async_copy(bufs[sl], xs_hbm.at[pidx_v.at[cb]], sems[sl])
        sc.wait()
        nxt = cb + 1
        if nxt < nch:
            pend[nxt % 2] = pltpu.async_copy(
                hidden_hbm.at[tok_v.at[pl.ds(nxt * RCHD, RCHD)]],
                bufs[nxt % 2], sems[nxt % 2])

    for nbc in range(NBPAD // 16):
        blk = (_vi(nbc * 16) + lanes) * _vi(B)
        cntv = _vi(0)
        for e in range(E):
            end_e = _lanesum(jnp.where(lanes == _vi(e), ends, _vi(0)))
            cntv = cntv + jnp.where(blk >= end_e, _vi(1), _vi(0))
        be_v[pl.ds(nbc * 16, 16)] = jnp.minimum(cntv, _vi(E - 1))

    @pl.when(wid == 0)
    def _publish():
        pltpu.sync_copy(be_v, be_hbm)


@functools.partial(
    pl.kernel, mesh=_mesh,
    out_type=jax.ShapeDtypeStruct((T, H), jnp.float32),
    scratch_types=[pltpu.VMEM((CHUNK,), jnp.float32),
                   pltpu.VMEM((CHUNK,), jnp.int32),
                   pltpu.VMEM((RCH, H), jnp.float32),
                   pltpu.VMEM((RCH // 2, H), jnp.float32),
                   pltpu.SemaphoreType.DMA])
def _sc_combine(y_hbm, pos_hbm, p_hbm, out_hbm, p_v, pos_v, rows_v, out_v,
                sem):
    wid = _wid()
    lanes = _lanes()
    abase = wid * CHUNK
    tbase = wid * TOKW
    pltpu.sync_copy(pos_hbm.at[pl.ds(abase, CHUNK)], pos_v)
    pltpu.sync_copy(p_hbm.at[pl.ds(abase, CHUNK)], p_v)
    for cb in range(CHUNK // RCH):
        pltpu.async_copy(
            y_hbm.at[pos_v.at[pl.ds(cb * RCH, RCH)]], rows_v, sem).wait()
        for i in range(RCH // 2):
            off = cb * RCH + 2 * i
            pv = p_v[pl.ds((off // 16) * 16, 16)]
            l0 = off % 16
            p0 = _lgather(pv, _vi(l0))
            p1 = _lgather(pv, _vi(l0 + 1))

            def qbody(q, _, i=i, p0=p0, p1=p1):
                for u in range(4):
                    sl = pl.ds((q * 4 + u) * 16, 16)
                    r0 = rows_v[2 * i, sl]
                    r1 = rows_v[2 * i + 1, sl]
                    out_v[i, sl] = p0 * r0 + p1 * r1
                return 0

            lax.fori_loop(0, H // 64, qbody, 0)
        pltpu.sync_copy(out_v, out_hbm.at[pl.ds(tbase + cb * (RCH // 2),
                                                RCH // 2)])


def _gelu_exact(x):
    return 0.5 * x * (1.0 + lax.erf(x * _INV_SQRT2))


def _mlp_body(be_ref, x_ref, w1_ref, b1_ref, w2_ref, b2_ref, y_ref):
    x = x_ref[...].astype(jnp.bfloat16)
    h = jnp.dot(x, w1_ref[0], preferred_element_type=jnp.float32) + b1_ref[0]
    h = _gelu_exact(h).astype(jnp.bfloat16)
    y_ref[...] = (jnp.dot(h, w2_ref[0], preferred_element_type=jnp.float32)
                  + b2_ref[0])


def _grouped_mlp(x_sorted, block_expert, W1b, b1r, W2b, b2r):
    grid_spec = pltpu.PrefetchScalarGridSpec(
        num_scalar_prefetch=1,
        grid=(NB,),
        in_specs=[
            pl.BlockSpec((B, H), lambda i, be: (i, 0)),
            pl.BlockSpec((1, H, F), lambda i, be: (be[i], 0, 0)),
            pl.BlockSpec((1, 1, F), lambda i, be: (be[i], 0, 0)),
            pl.BlockSpec((1, F, H), lambda i, be: (be[i], 0, 0)),
            pl.BlockSpec((1, 1, H), lambda i, be: (be[i], 0, 0)),
        ],
        out_specs=pl.BlockSpec((B, H), lambda i, be: (i, 0)),
    )
    return pl.pallas_call(
        _mlp_body,
        grid_spec=grid_spec,
        out_shape=jax.ShapeDtypeStruct((P, H), jnp.float32),
        compiler_params=pltpu.CompilerParams(
            dimension_semantics=("arbitrary",),
        ),
    )(block_expert, x_sorted, W1b, b1r, W2b, b2r)


@jax.jit
def _moe(hidden_flat, probs, indices, W1, b1, W2, b2):
    eid = indices.astype(jnp.int32).reshape(A)
    p_flat = probs.reshape(A)
    W1b = W1.astype(jnp.bfloat16)
    W2b = W2.astype(jnp.bfloat16)
    b1r = b1.reshape(E, 1, F)
    b2r = b2.reshape(E, 1, H)
    hist = _sc_hist(eid)
    pos, be, x_sorted = _sc_dispatch(eid, hist.reshape(NW * 16), hidden_flat)
    y_all = _grouped_mlp(x_sorted, be, W1b, b1r, W2b, b2r)
    return _sc_combine(y_all, pos, p_flat)


def kernel(hidden_flat, probs, indices, W1, b1, W2, b2):
    return _moe(hidden_flat, probs, indices, W1, b1, W2, b2)


# speedup vs baseline: 4.4663x; 1.2040x over previous
import functools

import jax
import jax.numpy as jnp
from jax import lax
from jax.experimental import pallas as pl
from jax.experimental.pallas import tpu as pltpu
from jax.experimental.pallas import tpu_sc as plsc

E, T, H, F, K = 8, 4096, 1024, 4096, 2
A = T * K
B = 512
P = A + E * B
NB = P // B
NBPAD = 32

NW = 32
CHUNK = A // NW
RCH = 64
RCHD = 16
NSLOT = 4
TOKW = T // NW

_INV_SQRT2 = 0.7071067811865476

_mesh = plsc.VectorSubcoreMesh(core_axis_name="c", subcore_axis_name="s")


def _wid():
    return lax.axis_index("s") * 2 + lax.axis_index("c")


def _lanes():
    return lax.broadcasted_iota(jnp.int32, (16,), 0)


def _vi(x):
    return jnp.full((16,), x, jnp.int32)


def _vf(x):
    return jnp.full((16,), x, jnp.float32)


_GDN = lax.GatherDimensionNumbers(
    offset_dims=(), collapsed_slice_dims=(0,), start_index_map=(0,))


def _lgather(x, idx):
    return lax.gather(x, idx[:, None], _GDN, (1,),
                      mode=lax.GatherScatterMode.PROMISE_IN_BOUNDS)


def _lanesum(x):
    lanes = _lanes()
    for d in (1, 2, 4, 8):
        x = x + _lgather(x, lanes ^ _vi(d))
    return x


def _prefix_incl(x):
    lanes = _lanes()
    zero = jnp.zeros((16,), x.dtype)
    for d in (1, 2, 4, 8):
        shifted = _lgather(x, jnp.maximum(lanes - _vi(d), _vi(0)))
        x = x + jnp.where(lanes >= _vi(d), shifted, zero)
    return x


@functools.partial(
    pl.kernel, mesh=_mesh,
    out_type=jax.ShapeDtypeStruct((NW, 16), jnp.int32),
    scratch_types=[pltpu.VMEM((CHUNK,), jnp.int32),
                   pltpu.VMEM((16,), jnp.int32)])
def _sc_hist(eid_hbm, hist_hbm, eid_v, cnt_v):
    wid = _wid()
    pltpu.sync_copy(eid_hbm.at[pl.ds(wid * CHUNK, CHUNK)], eid_v)
    lanes = _lanes()

    def body(j, cnt):
        v = eid_v[pl.ds(j * 16, 16)]
        for e in range(E):
            pc = _lanesum(jnp.where(v == _vi(e), _vi(1), _vi(0)))
            cnt = cnt + jnp.where(lanes == _vi(e), pc, _vi(0))
        return cnt

    cnt = lax.fori_loop(0, CHUNK // 16, body, jnp.zeros((16,), jnp.int32))
    cnt_v[...] = cnt
    pltpu.sync_copy(cnt_v, hist_hbm.at[wid])


@functools.partial(
    pl.kernel, mesh=_mesh,
    out_type=(jax.ShapeDtypeStruct((A,), jnp.int32),
              jax.ShapeDtypeStruct((NBPAD,), jnp.int32),
              jax.ShapeDtypeStruct((NBPAD,), jnp.int32),
              jax.ShapeDtypeStruct((P, H), jnp.float32)),
    scratch_types=[pltpu.VMEM((CHUNK,), jnp.int32),
                   pltpu.VMEM((NW * 16,), jnp.int32),
                   pltpu.VMEM((CHUNK,), jnp.int32),
                   pltpu.VMEM((CHUNK,), jnp.int32),
                   pltpu.VMEM((CHUNK // RCHD, RCHD), jnp.int32),
                   pltpu.VMEM((NSLOT * RCHD, H), jnp.float32),
                   pltpu.VMEM((NBPAD,), jnp.int32),
                   pltpu.VMEM((NBPAD,), jnp.int32),
                   pltpu.SemaphoreType.DMA,
                   pltpu.SemaphoreType.DMA,
                   pltpu.SemaphoreType.DMA,
                   pltpu.SemaphoreType.DMA])
def _sc_dispatch(eid_hbm, hist_hbm, hidden_hbm, pos_hbm, be_hbm, valid_hbm,
                 xs_hbm, eid_v, hist_v, pos_v, tok_v, pidx_v, rows_all, be_v,
                 valid_v, sem0, sem1, sem2, sem3):
    wid = _wid()
    lanes = _lanes()
    pltpu.sync_copy(eid_hbm.at[pl.ds(wid * CHUNK, CHUNK)], eid_v)
    pltpu.sync_copy(hist_hbm, hist_v)

    def hbody(w, carry):
        tot, mine = carry
        row = hist_v[pl.ds(w * 16, 16)]
        sel = jnp.where(w < wid, 1, 0)
        return tot + row, mine + row * _vi(sel)

    zeros16 = jnp.zeros((16,), jnp.int32)
    tot, mine = lax.fori_loop(0, NW, hbody, (zeros16, zeros16))

    padded = ((tot + _vi(B - 1)) >> 9) << 9
    ends = _prefix_incl(padded)
    bases = ends - padded
    cur0 = bases + mine

    def pbody(j, cur):
        v = eid_v[pl.ds(j * 16, 16)]
        pos = _vi(0)
        for e in range(E):
            m = v == _vi(e)
            m01 = jnp.where(m, _vi(1), _vi(0))
            r = _prefix_incl(m01)
            base_e = _lanesum(jnp.where(lanes == _vi(e), cur, _vi(0)))
            pos = jnp.where(m, base_e - _vi(1) + r, pos)
            pc = _lgather(r, _vi(15))
            cur = cur + jnp.where(lanes == _vi(e), pc, _vi(0))
        pos_v[pl.ds(j * 16, 16)] = pos
        return cur

    lax.fori_loop(0, CHUNK // 16, pbody, cur0)
    pltpu.sync_copy(pos_v, pos_hbm.at[pl.ds(wid * CHUNK, CHUNK)])

    def tbody(j, _):
        base = wid * CHUNK + j * 16
        tok_v[pl.ds(j * 16, 16)] = (_vi(base) + lanes) >> _vi(1)
        return 0

    lax.fori_loop(0, CHUNK // 16, tbody, 0)

    for cb in range(CHUNK // RCHD):
        for g in range(RCHD // 16):
            pidx_v[cb, pl.ds(g * 16, 16)] = pos_v[pl.ds(cb * RCHD + g * 16,
                                                        16)]

    nch = CHUNK // RCHD
    sems = (sem0, sem1, sem2, sem3)

    def _slot(s):
        return rows_all.at[pl.ds(s * RCHD, RCHD)]

    def _gather(cb, s):
        return pltpu.async_copy(
            hidden_hbm.at[tok_v.at[pl.ds(cb * RCHD, RCHD)]],
            _slot(s), sems[s])

    gpend = {0: _gather(0, 0), 1: _gather(1, 1)}
    spend = [None] * NSLOT
    for cb in range(nch):
        s = cb % NSLOT
        gpend.pop(cb).wait()
        spend[s] = pltpu.async_copy(_slot(s), xs_hbm.at[pidx_v.at[cb]],
                                    sems[s])
        la = cb + 2
        if la < nch:
            sl = la % NSLOT
            if spend[sl] is not None:
                spend[sl].wait()
                spend[sl] = None
            gpend[la] = _gather(la, sl)
    for s in range(NSLOT):
        if spend[s] is not None:
            spend[s].wait()

    rends = bases + tot
    for nbc in range(NBPAD // 16):
        blk = (_vi(nbc * 16) + lanes) * _vi(B)
        cntv = _vi(0)
        for e in range(E):
            end_e = _lanesum(jnp.where(lanes == _vi(e), ends, _vi(0)))
            cntv = cntv + jnp.where(blk >= end_e, _vi(1), _vi(0))
        be16 = jnp.minimum(cntv, _vi(E - 1))
        be_v[pl.ds(nbc * 16, 16)] = be16
        rend_g = _lgather(rends, be16)
        valid_v[pl.ds(nbc * 16, 16)] = jnp.where(blk < rend_g, _vi(1),
                                                 _vi(0))

    @pl.when(wid == 0)
    def _publish():
        pltpu.sync_copy(be_v, be_hbm)
        pltpu.sync_copy(valid_v, valid_hbm)


@functools.partial(
    pl.kernel, mesh=_mesh,
    out_type=jax.ShapeDtypeStruct((T, H), jnp.float32),
    scratch_types=[pltpu.VMEM((CHUNK,), jnp.float32),
                   pltpu.VMEM((CHUNK,), jnp.int32),
                   pltpu.VMEM((RCH, H), jnp.float32),
                   pltpu.VMEM((RCH // 2, H), jnp.float32),
                   pltpu.SemaphoreType.DMA])
def _sc_combine(y_hbm, pos_hbm, p_hbm, out_hbm, p_v, pos_v, rows_v, out_v,
                sem):
    wid = _wid()
    lanes = _lanes()
    abase = wid * CHUNK
    tbase = wid * TOKW
    pltpu.sync_copy(pos_hbm.at[pl.ds(abase, CHUNK)], pos_v)
    pltpu.sync_copy(p_hbm.at[pl.ds(abase, CHUNK)], p_v)
    for cb in range(CHUNK // RCH):
        pltpu.async_copy(
            y_hbm.at[pos_v.at[pl.ds(cb * RCH, RCH)]], rows_v, sem).wait()
        for sg in range(RCH // 16):
            pv = p_v[pl.ds(cb * RCH + sg * 16, 16)]
            p0 = [_lgather(pv, _vi(2 * t)) for t in range(8)]
            p1 = [_lgather(pv, _vi(2 * t + 1)) for t in range(8)]

            def qbody(q, _, sg=sg, p0=p0, p1=p1):
                sl = pl.ds(q * 16, 16)
                for t in range(8):
                    r0 = rows_v[sg * 16 + 2 * t, sl]
                    r1 = rows_v[sg * 16 + 2 * t + 1, sl]
                    out_v[sg * 8 + t, sl] = p0[t] * r0 + p1[t] * r1
                return 0

            lax.fori_loop(0, H // 16, qbody, 0)
        pltpu.sync_copy(out_v, out_hbm.at[pl.ds(tbase + cb * (RCH // 2),
                                                RCH // 2)])


def _gelu_exact(x):
    return 0.5 * x * (1.0 + lax.erf(x * _INV_SQRT2))


def _mlp_body(be_ref, valid_ref, x_ref, w1_ref, b1_ref, w2_ref, b2_ref,
              y_ref):
    i = pl.program_id(0)

    @pl.when(valid_ref[i] != 0)
    def _compute():
        x = x_ref[...].astype(jnp.bfloat16)
        h = (jnp.dot(x, w1_ref[0], preferred_element_type=jnp.float32)
             + b1_ref[0])
        h = _gelu_exact(h).astype(jnp.bfloat16)
        y_ref[...] = (jnp.dot(h, w2_ref[0],
                              preferred_element_type=jnp.float32)
                      + b2_ref[0])


def _grouped_mlp(x_sorted, block_expert, block_valid, W1b, b1r, W2b, b2r):
    grid_spec = pltpu.PrefetchScalarGridSpec(
        num_scalar_prefetch=2,
        grid=(NB,),
        in_specs=[
            pl.BlockSpec((B, H), lambda i, be, va: (i, 0)),
            pl.BlockSpec((1, H, F), lambda i, be, va: (be[i], 0, 0)),
            pl.BlockSpec((1, 1, F), lambda i, be, va: (be[i], 0, 0)),
            pl.BlockSpec((1, F, H), lambda i, be, va: (be[i], 0, 0)),
            pl.BlockSpec((1, 1, H), lambda i, be, va: (be[i], 0, 0)),
        ],
        out_specs=pl.BlockSpec((B, H), lambda i, be, va: (i, 0)),
    )
    return pl.pallas_call(
        _mlp_body,
        grid_spec=grid_spec,
        out_shape=jax.ShapeDtypeStruct((P, H), jnp.float32),
        compiler_params=pltpu.CompilerParams(
            dimension_semantics=("arbitrary",),
        ),
    )(block_expert, block_valid, x_sorted, W1b, b1r, W2b, b2r)


@jax.jit
def _moe(hidden_flat, probs, indices, W1, b1, W2, b2):
    eid = indices.astype(jnp.int32).reshape(A)
    p_flat = probs.reshape(A)
    W1b = W1.astype(jnp.bfloat16)
    W2b = W2.astype(jnp.bfloat16)
    b1r = b1.reshape(E, 1, F)
    b2r = b2.reshape(E, 1, H)
    hist = _sc_hist(eid)
    pos, be, valid, x_sorted = _sc_dispatch(eid, hist.reshape(NW * 16),
                                            hidden_flat)
    y_all = _grouped_mlp(x_sorted, be, valid, W1b, b1r, W2b, b2r)
    return _sc_combine(y_all, pos, p_flat)


def kernel(hidden_flat, probs, indices, W1, b1, W2, b2):
    return _moe(hidden_flat, probs, indices, W1, b1, W2, b2)


# speedup vs baseline: 4.8013x; 1.0750x over previous
import functools

import jax
import jax.numpy as jnp
from jax import lax
from jax.experimental import pallas as pl
from jax.experimental.pallas import tpu as pltpu
from jax.experimental.pallas import tpu_sc as plsc

E, T, H, F, K = 8, 4096, 1024, 4096, 2
A = T * K
B = 512
P = A + E * B
NB = P // B
NBPAD = 32

NW = 32
CHUNK = A // NW
RCH = 64
RCHD = 16
NSLOT = 4
TOKW = T // NW

_INV_SQRT2 = 0.7071067811865476

_mesh = plsc.VectorSubcoreMesh(core_axis_name="c", subcore_axis_name="s")


def _wid():
    return lax.axis_index("s") * 2 + lax.axis_index("c")


def _lanes():
    return lax.broadcasted_iota(jnp.int32, (16,), 0)


def _vi(x):
    return jnp.full((16,), x, jnp.int32)


def _vf(x):
    return jnp.full((16,), x, jnp.float32)


_GDN = lax.GatherDimensionNumbers(
    offset_dims=(), collapsed_slice_dims=(0,), start_index_map=(0,))


def _lgather(x, idx):
    return lax.gather(x, idx[:, None], _GDN, (1,),
                      mode=lax.GatherScatterMode.PROMISE_IN_BOUNDS)


def _lanesum(x):
    lanes = _lanes()
    for d in (1, 2, 4, 8):
        x = x + _lgather(x, lanes ^ _vi(d))
    return x


def _prefix_incl(x):
    lanes = _lanes()
    zero = jnp.zeros((16,), x.dtype)
    for d in (1, 2, 4, 8):
        shifted = _lgather(x, jnp.maximum(lanes - _vi(d), _vi(0)))
        x = x + jnp.where(lanes >= _vi(d), shifted, zero)
    return x


@functools.partial(
    pl.kernel, mesh=_mesh,
    out_type=jax.ShapeDtypeStruct((NW, 16), jnp.int32),
    scratch_types=[pltpu.VMEM((CHUNK,), jnp.int32),
                   pltpu.VMEM((16,), jnp.int32)])
def _sc_hist(eid_hbm, hist_hbm, eid_v, cnt_v):
    wid = _wid()
    pltpu.sync_copy(eid_hbm.at[pl.ds(wid * CHUNK, CHUNK)], eid_v)
    lanes = _lanes()

    def body(j, cnt):
        v = eid_v[pl.ds(j * 16, 16)]
        for e in range(E):
            pc = _lanesum(jnp.where(v == _vi(e), _vi(1), _vi(0)))
            cnt = cnt + jnp.where(lanes == _vi(e), pc, _vi(0))
        return cnt

    cnt = lax.fori_loop(0, CHUNK // 16, body, jnp.zeros((16,), jnp.int32))
    cnt_v[...] = cnt
    pltpu.sync_copy(cnt_v, hist_hbm.at[wid])


@functools.partial(
    pl.kernel, mesh=_mesh,
    out_type=(jax.ShapeDtypeStruct((A,), jnp.int32),
              jax.ShapeDtypeStruct((NBPAD,), jnp.int32),
              jax.ShapeDtypeStruct((NBPAD,), jnp.int32),
              jax.ShapeDtypeStruct((P, H), jnp.float32)),
    scratch_types=[pltpu.VMEM((CHUNK,), jnp.int32),
                   pltpu.VMEM((NW * 16,), jnp.int32),
                   pltpu.VMEM((CHUNK,), jnp.int32),
                   pltpu.VMEM((CHUNK,), jnp.int32),
                   pltpu.VMEM((CHUNK // RCHD, RCHD), jnp.int32),
                   pltpu.VMEM((NSLOT * RCHD, H), jnp.float32),
                   pltpu.VMEM((NBPAD,), jnp.int32),
                   pltpu.VMEM((NBPAD,), jnp.int32),
                   pltpu.SemaphoreType.DMA,
                   pltpu.SemaphoreType.DMA,
                   pltpu.SemaphoreType.DMA,
                   pltpu.SemaphoreType.DMA])
def _sc_dispatch(eid_hbm, hist_hbm, hidden_hbm, pos_hbm, be_hbm, valid_hbm,
                 xs_hbm, eid_v, hist_v, pos_v, tok_v, pidx_v, rows_all, be_v,
                 valid_v, sem0, sem1, sem2, sem3):
    wid = _wid()
    lanes = _lanes()
    pltpu.sync_copy(eid_hbm.at[pl.ds(wid * CHUNK, CHUNK)], eid_v)
    pltpu.sync_copy(hist_hbm, hist_v)

    def hbody(w, carry):
        tot, mine = carry
        row = hist_v[pl.ds(w * 16, 16)]
        sel = jnp.where(w < wid, 1, 0)
        return tot + row, mine + row * _vi(sel)

    zeros16 = jnp.zeros((16,), jnp.int32)
    tot, mine = lax.fori_loop(0, NW, hbody, (zeros16, zeros16))

    padded = ((tot + _vi(B - 1)) >> 9) << 9
    ends = _prefix_incl(padded)
    bases = ends - padded
    cur0 = bases + mine

    def pbody(j, cur):
        v = eid_v[pl.ds(j * 16, 16)]
        pos = _vi(0)
        for e in range(E):
            m = v == _vi(e)
            m01 = jnp.where(m, _vi(1), _vi(0))
            r = _prefix_incl(m01)
            base_e = _lanesum(jnp.where(lanes == _vi(e), cur, _vi(0)))
            pos = jnp.where(m, base_e - _vi(1) + r, pos)
            pc = _lgather(r, _vi(15))
            cur = cur + jnp.where(lanes == _vi(e), pc, _vi(0))
        pos_v[pl.ds(j * 16, 16)] = pos
        return cur

    lax.fori_loop(0, CHUNK // 16, pbody, cur0)
    pltpu.sync_copy(pos_v, pos_hbm.at[pl.ds(wid * CHUNK, CHUNK)])

    def tbody(j, _):
        base = wid * CHUNK + j * 16
        tok_v[pl.ds(j * 16, 16)] = (_vi(base) + lanes) >> _vi(1)
        return 0

    lax.fori_loop(0, CHUNK // 16, tbody, 0)

    for cb in range(CHUNK // RCHD):
        for g in range(RCHD // 16):
            pidx_v[cb, pl.ds(g * 16, 16)] = pos_v[pl.ds(cb * RCHD + g * 16,
                                                        16)]

    nch = CHUNK // RCHD
    sems = (sem0, sem1, sem2, sem3)

    def _slot(s):
        return rows_all.at[pl.ds(s * RCHD, RCHD)]

    def _gather(cb, s):
        return pltpu.async_copy(
            hidden_hbm.at[tok_v.at[pl.ds(cb * RCHD, RCHD)]],
            _slot(s), sems[s])

    gpend = {0: _gather(0, 0), 1: _gather(1, 1)}
    spend = [None] * NSLOT
    for cb in range(nch):
        s = cb % NSLOT
        gpend.pop(cb).wait()
        spend[s] = pltpu.async_copy(_slot(s), xs_hbm.at[pidx_v.at[cb]],
                                    sems[s])
        la = cb + 2
        if la < nch:
            sl = la % NSLOT
            if spend[sl] is not None:
                spend[sl].wait()
                spend[sl] = None
            gpend[la] = _gather(la, sl)
    for s in range(NSLOT):
        if spend[s] is not None:
            spend[s].wait()

    rends = bases + tot
    for nbc in range(NBPAD // 16):
        blk = (_vi(nbc * 16) + lanes) * _vi(B)
        cntv = _vi(0)
        for e in range(E):
            end_e = _lanesum(jnp.where(lanes == _vi(e), ends, _vi(0)))
            cntv = cntv + jnp.where(blk >= end_e, _vi(1), _vi(0))
        be16 = jnp.minimum(cntv, _vi(E - 1))
        be_v[pl.ds(nbc * 16, 16)] = be16
        rend_g = _lgather(rends, be16)
        valid_v[pl.ds(nbc * 16, 16)] = jnp.where(blk < rend_g, _vi(1),
                                                 _vi(0))

    @pl.when(wid == 0)
    def _publish():
        pltpu.sync_copy(be_v, be_hbm)
        pltpu.sync_copy(valid_v, valid_hbm)


@functools.partial(
    pl.kernel, mesh=_mesh,
    out_type=jax.ShapeDtypeStruct((T, H), jnp.float32),
    scratch_types=[pltpu.VMEM((CHUNK,), jnp.float32),
                   pltpu.VMEM((CHUNK,), jnp.int32),
                   pltpu.VMEM((RCH, H), jnp.float32),
                   pltpu.VMEM((RCH // 2, H), jnp.float32),
                   pltpu.SemaphoreType.DMA])
def _sc_combine(y_hbm, pos_hbm, p_hbm, out_hbm, p_v, pos_v, rows_v, out_v,
                sem):
    wid = _wid()
    lanes = _lanes()
    abase = wid * CHUNK
    tbase = wid * TOKW
    pltpu.sync_copy(pos_hbm.at[pl.ds(abase, CHUNK)], pos_v)
    pltpu.sync_copy(p_hbm.at[pl.ds(abase, CHUNK)], p_v)
    for cb in range(CHUNK // RCH):
        pltpu.async_copy(
            y_hbm.at[pos_v.at[pl.ds(cb * RCH, RCH)]], rows_v, sem).wait()
        for sg in range(RCH // 16):
            pv = p_v[pl.ds(cb * RCH + sg * 16, 16)]
            p0 = [_lgather(pv, _vi(2 * t)) for t in range(8)]
            p1 = [_lgather(pv, _vi(2 * t + 1)) for t in range(8)]

            def qbody(q, _, sg=sg, p0=p0, p1=p1):
                sl = pl.ds(q * 16, 16)
                for t in range(8):
                    r0 = rows_v[sg * 16 + 2 * t, sl]
                    r1 = rows_v[sg * 16 + 2 * t + 1, sl]
                    out_v[sg * 8 + t, sl] = p0[t] * r0 + p1[t] * r1
                return 0

            lax.fori_loop(0, H // 16, qbody, 0)
        pltpu.sync_copy(out_v, out_hbm.at[pl.ds(tbase + cb * (RCH // 2),
                                                RCH // 2)])


def _gelu_exact(x):
    return 0.5 * x * (1.0 + lax.erf(x * _INV_SQRT2))


def _fc1_body(be_ref, valid_ref, x_ref, w1_ref, b1_ref, h_ref):
    i = pl.program_id(0)

    @pl.when(valid_ref[i] != 0)
    def _compute():
        x = x_ref[...].astype(jnp.bfloat16)
        w1 = w1_ref[0].astype(jnp.bfloat16)
        h = (jnp.dot(x, w1, preferred_element_type=jnp.float32)
             + b1_ref[0])
        h_ref[...] = _gelu_exact(h).astype(jnp.bfloat16)


def _fc2_body(be_ref, valid_ref, h_ref, w2_ref, b2_ref, y_ref):
    i = pl.program_id(0)

    @pl.when(valid_ref[i] != 0)
    def _compute():
        w2 = w2_ref[0].astype(jnp.bfloat16)
        y_ref[...] = (jnp.dot(h_ref[...], w2,
                              preferred_element_type=jnp.float32)
                      + b2_ref[0])


def _grouped_mlp(x_sorted, block_expert, block_valid, W1, b1r, W2, b2r):
    fc1_spec = pltpu.PrefetchScalarGridSpec(
        num_scalar_prefetch=2,
        grid=(NB,),
        in_specs=[
            pl.BlockSpec((B, H), lambda i, be, va: (i, 0)),
            pl.BlockSpec((1, H, F), lambda i, be, va: (be[i], 0, 0)),
            pl.BlockSpec((1, 1, F), lambda i, be, va: (be[i], 0, 0)),
        ],
        out_specs=pl.BlockSpec((B, F), lambda i, be, va: (i, 0)),
    )
    h_all = pl.pallas_call(
        _fc1_body,
        grid_spec=fc1_spec,
        out_shape=jax.ShapeDtypeStruct((P, F), jnp.bfloat16),
        compiler_params=pltpu.CompilerParams(
            dimension_semantics=("arbitrary",),
        ),
    )(block_expert, block_valid, x_sorted, W1, b1r)
    fc2_spec = pltpu.PrefetchScalarGridSpec(
        num_scalar_prefetch=2,
        grid=(NB,),
        in_specs=[
            pl.BlockSpec((B, F), lambda i, be, va: (i, 0)),
            pl.BlockSpec((1, F, H), lambda i, be, va: (be[i], 0, 0)),
            pl.BlockSpec((1, 1, H), lambda i, be, va: (be[i], 0, 0)),
        ],
        out_specs=pl.BlockSpec((B, H), lambda i, be, va: (i, 0)),
    )
    return pl.pallas_call(
        _fc2_body,
        grid_spec=fc2_spec,
        out_shape=jax.ShapeDtypeStruct((P, H), jnp.float32),
        compiler_params=pltpu.CompilerParams(
            dimension_semantics=("arbitrary",),
        ),
    )(block_expert, block_valid, h_all, W2, b2r)


@jax.jit
def _moe(hidden_flat, probs, indices, W1, b1, W2, b2):
    eid = indices.astype(jnp.int32).reshape(A)
    p_flat = probs.reshape(A)
    b1r = b1.reshape(E, 1, F)
    b2r = b2.reshape(E, 1, H)
    hist = _sc_hist(eid)
    pos, be, valid, x_sorted = _sc_dispatch(eid, hist.reshape(NW * 16),
                                            hidden_flat)
    y_all = _grouped_mlp(x_sorted, be, valid, W1, b1r, W2, b2r)
    return _sc_combine(y_all, pos, p_flat)


def kernel(hidden_flat, probs, indices, W1, b1, W2, b2):
    return _moe(hidden_flat, probs, indices, W1, b1, W2, b2)
